# baseline (device time: 188647 ns/iter reference)
import jax
import jax.numpy as jnp
from jax import lax
from jax.experimental import pallas as pl
from jax.experimental.pallas import tpu as pltpu

N_DEV = 8
SQ = 2048
SKV = 2048
D_MODEL = 1024
DH = 128
H_LOCAL = 8
WIN = 128
QBLK = 256
KSPAN = 512
SCALE = 0.08838834764831843

_AG_GROUPS = (
    (0, 384, (2, 1, 0)),
    (384, 384, (0, 2, 1)),
    (768, 256, (1, 0, 2)),
)
_FLIP = (4, 2, 1)


def _coords(p):
    q = lax.rem(p, 4)
    zb = p // 4
    xb = lax.rem((q + 1) // 2, 2)
    yb = q // 2
    return xb, yb, zb


def _nest(p):
    xb, yb, zb = _coords(p)
    return 4 * xb + 2 * yb + zb


def _body(
    x_ref, wq_ref, k_ref, v_ref, wo_ref, out_ref,
    comm_ref, rs_send, rs_recv, ag_send, ag_recv, ready_sem,
):
    s = pl.program_id(0)
    h = pl.program_id(1)
    pos = lax.axis_index("i")
    right = lax.rem(pos + 1, N_DEV)
    left = lax.rem(pos + N_DEV - 1, N_DEV)

    cs = _nest(lax.rem(pos - s - 1 + 2 * N_DEV, N_DEV))
    rows = cs * QBLK

    qblk = (
        jnp.dot(
            x_ref[pl.ds(rows, QBLK), :],
            wq_ref[:, pl.ds(h * DH, DH)],
            preferred_element_type=jnp.float32,
        )
        * SCALE
    ).astype(jnp.bfloat16)
    start = jnp.clip(cs * 2 - 1, 0, (SKV - KSPAN) // 128) * 128
    kblk = k_ref[pl.ds(start, KSPAN), pl.ds(h * DH, DH)]
    sc = lax.dot_general(
        qblk, kblk, (((1,), (1,)), ((), ())),
        preferred_element_type=jnp.float32,
    )
    qi = rows + lax.broadcasted_iota(jnp.int32, (QBLK, KSPAN), 0)
    ki = start + lax.broadcasted_iota(jnp.int32, (QBLK, KSPAN), 1)
    sc = jnp.where(jnp.abs(qi - ki) <= WIN, sc, -1e9)
    m = jnp.max(sc, axis=1, keepdims=True)
    w = jnp.exp(sc - m)
    denom = jnp.sum(w, axis=1, keepdims=True)
    vblk = v_ref[pl.ds(start, KSPAN), pl.ds(h * DH, DH)]
    ctx = (
        jnp.dot(
            w.astype(jnp.bfloat16), vblk, preferred_element_type=jnp.float32
        )
        / denom
    ).astype(jnp.bfloat16)
    contrib = jnp.dot(
        ctx, wo_ref[pl.ds(h * DH, DH), :], preferred_element_type=jnp.float32
    )

    @pl.when(h == 0)
    def _():
        out_ref[pl.ds(rows, QBLK), :] = contrib

    @pl.when(h != 0)
    def _():
        out_ref[pl.ds(rows, QBLK), :] += contrib

    @pl.when(h == H_LOCAL - 1)
    def _():
        sm1 = jnp.maximum(s - 1, 0)

        def ring_desc(idx, src_rows):
            return pltpu.make_async_remote_copy(
                src_ref=out_ref.at[pl.ds(src_rows, QBLK), :],
                dst_ref=comm_ref.at[pl.ds(idx * QBLK, QBLK), :],
                send_sem=rs_send.at[idx],
                recv_sem=rs_recv.at[idx],
                device_id=(right,),
                device_id_type=pl.DeviceIdType.MESH,
            )

        @pl.when(s > 0)
        def _():
            ring_desc(sm1, rows).wait_recv()
            out_ref[pl.ds(rows, QBLK), :] += comm_ref[pl.ds(sm1 * QBLK, QBLK), :]

        @pl.when(s < N_DEV - 1)
        def _():
            ring_desc(s, rows).start()

        @pl.when(s > 0)
        def _():
            ring_desc(jnp.minimum(sm1, N_DEV - 2), rows).wait_send()

    @pl.when(jnp.logical_and(s == N_DEV - 1, h == H_LOCAL - 1))
    def _():
        xb, yb, zb = _coords(pos)
        bits = [xb, yb, zb]
        q4 = lax.rem(pos, 4)
        partners = [
            zb * 4 + jnp.bitwise_xor(q4, 1),
            zb * 4 + (3 - q4),
            lax.rem(pos + 4, N_DEV),
        ]
        c_own = _nest(pos)

        for d in range(3):
            pl.semaphore_signal(
                ready_sem.at[d],
                inc=1,
                device_id=(partners[d],),
                device_id_type=pl.DeviceIdType.MESH,
            )
        for d in range(3):
            pl.semaphore_wait(ready_sem.at[d], 1)

        for t in range(3):
            descs = []
            for gi, (coff, cw, order) in enumerate(_AG_GROUPS):
                d = order[t]
                masks = [0]
                for td in order[:t]:
                    masks = masks + [mk ^ _FLIP[td] for mk in masks]
                for j, mk in enumerate(masks):
                    cc = jnp.bitwise_xor(c_own, mk)
                    r = cc * QBLK
                    rdma = pltpu.make_async_remote_copy(
                        src_ref=out_ref.at[pl.ds(r, QBLK), pl.ds(coff, cw)],
                        dst_ref=out_ref.at[pl.ds(r, QBLK), pl.ds(coff, cw)],
                        send_sem=ag_send.at[gi, t, j],
                        recv_sem=ag_recv.at[gi, t, j],
                        device_id=(partners[d],),
                        device_id_type=pl.DeviceIdType.MESH,
                    )
                    rdma.start()
                    descs.append(rdma)
            for rdma in descs:
                rdma.wait_recv()
                rdma.wait_send()


def kernel(x, Wq, K_ext, V_ext, Wo):
    pos = lax.axis_index("i")
    K = (
        lax.dynamic_slice_in_dim(K_ext[0], pos * H_LOCAL, H_LOCAL, axis=1)
        .reshape(SKV, H_LOCAL * DH)
        .astype(jnp.bfloat16)
    )
    V = (
        lax.dynamic_slice_in_dim(V_ext[0], pos * H_LOCAL, H_LOCAL, axis=1)
        .reshape(SKV, H_LOCAL * DH)
        .astype(jnp.bfloat16)
    )

    out = pl.pallas_call(
        _body,
        grid=(N_DEV, H_LOCAL),
        in_specs=[
            pl.BlockSpec((SQ, D_MODEL), lambda s, h: (0, 0)),
            pl.BlockSpec((D_MODEL, H_LOCAL * DH), lambda s, h: (0, 0)),
            pl.BlockSpec((SKV, H_LOCAL * DH), lambda s, h: (0, 0)),
            pl.BlockSpec((SKV, H_LOCAL * DH), lambda s, h: (0, 0)),
            pl.BlockSpec((H_LOCAL * DH, D_MODEL), lambda s, h: (0, 0)),
        ],
        out_specs=pl.BlockSpec((SQ, D_MODEL), lambda s, h: (0, 0)),
        out_shape=jax.ShapeDtypeStruct((SQ, D_MODEL), jnp.float32),
        scratch_shapes=[
            pltpu.VMEM(((N_DEV - 1) * QBLK, D_MODEL), jnp.float32),
            pltpu.SemaphoreType.DMA((N_DEV - 1,)),
            pltpu.SemaphoreType.DMA((N_DEV - 1,)),
            pltpu.SemaphoreType.DMA((3, 3, 4)),
            pltpu.SemaphoreType.DMA((3, 3, 4)),
            pltpu.SemaphoreType.REGULAR((3,)),
        ],
        compiler_params=pltpu.CompilerParams(
            dimension_semantics=("arbitrary", "arbitrary"),
            has_side_effects=True,
            vmem_limit_bytes=56 * 1024 * 1024,
        ),
    )(
        x[0].astype(jnp.bfloat16),
        Wq.astype(jnp.bfloat16),
        K,
        V,
        Wo.astype(jnp.bfloat16),
    )
    return out[None]


# device time: 186112 ns/iter; 1.0136x vs baseline; 1.0136x over previous
import jax
import jax.numpy as jnp
from jax import lax
from jax.experimental import pallas as pl
from jax.experimental.pallas import tpu as pltpu

N_DEV = 8
SQ = 2048
SKV = 2048
D_MODEL = 1024
DH = 128
H_LOCAL = 8
WIN = 128
QBLK = 256
KSPAN = 512
SCALE = 0.08838834764831843

_AG_GROUPS = (
    (0, 384, (2, 1, 0)),
    (384, 384, (0, 2, 1)),
    (768, 256, (1, 0, 2)),
)
_FLIP = (4, 2, 1)


def _coords(p):
    q = lax.rem(p, 4)
    zb = p // 4
    xb = lax.rem((q + 1) // 2, 2)
    yb = q // 2
    return xb, yb, zb


def _nest(p):
    xb, yb, zb = _coords(p)
    return 4 * xb + 2 * yb + zb


def _body(
    x_ref, wq_ref, k_ref, v_ref, wo_ref, out_ref,
    comm_ref, stage_ref, qbf_scr, ctx_scr,
    rs_send, rs_recv, ag_send, ag_recv, ready_sem,
):
    s = pl.program_id(0)
    pos = lax.axis_index("i")
    right = lax.rem(pos + 1, N_DEV)

    cs = _nest(lax.rem(pos - s - 1 + 2 * N_DEV, N_DEV))
    rows = cs * QBLK
    start = jnp.clip(cs * 2 - 1, 0, (SKV - KSPAN) // 128) * 128

    qbf_scr[...] = (
        jnp.dot(
            x_ref[pl.ds(rows, QBLK), :],
            wq_ref[...],
            preferred_element_type=jnp.float32,
        )
        * SCALE
    ).astype(jnp.bfloat16)

    qi = rows + lax.broadcasted_iota(jnp.int32, (QBLK, KSPAN), 0)
    ki = start + lax.broadcasted_iota(jnp.int32, (QBLK, KSPAN), 1)
    win_mask = jnp.abs(qi - ki) <= WIN

    def head(h, carry):
        qh = qbf_scr[:, pl.ds(h * DH, DH)]
        kblk = k_ref[pl.ds(start, KSPAN), pl.ds(h * DH, DH)]
        sc = lax.dot_general(
            qh, kblk, (((1,), (1,)), ((), ())),
            preferred_element_type=jnp.float32,
        )
        sc = jnp.where(win_mask, sc, -1e9)
        m = jnp.max(sc, axis=1, keepdims=True)
        w = jnp.exp(sc - m)
        denom = jnp.sum(w, axis=1, keepdims=True)
        vblk = v_ref[pl.ds(start, KSPAN), pl.ds(h * DH, DH)]
        ctx = (
            jnp.dot(
                w.astype(jnp.bfloat16), vblk,
                preferred_element_type=jnp.float32,
            )
            / denom
        )
        ctx_scr[:, pl.ds(h * DH, DH)] = ctx.astype(jnp.bfloat16)
        return carry

    lax.fori_loop(0, H_LOCAL, head, 0)

    out_ref[pl.ds(rows, QBLK), :] = jnp.dot(
        ctx_scr[...], wo_ref[...], preferred_element_type=jnp.float32
    )

    sm1 = jnp.maximum(s - 1, 0)

    def ring_desc(idx):
        return pltpu.make_async_remote_copy(
            src_ref=stage_ref.at[pl.ds(idx * QBLK, QBLK), :],
            dst_ref=comm_ref.at[pl.ds(idx * QBLK, QBLK), :],
            send_sem=rs_send.at[idx],
            recv_sem=rs_recv.at[idx],
            device_id=(right,),
            device_id_type=pl.DeviceIdType.MESH,
        )

    @pl.when(s > 0)
    def _():
        ring_desc(sm1).wait_recv()
        out_ref[pl.ds(rows, QBLK), :] += comm_ref[pl.ds(sm1 * QBLK, QBLK), :]

    @pl.when(s < N_DEV - 1)
    def _():
        stage_ref[pl.ds(s * QBLK, QBLK), :] = out_ref[pl.ds(rows, QBLK), :]
        ring_desc(s).start()

    @pl.when(s > 0)
    def _():
        ring_desc(jnp.minimum(sm1, N_DEV - 2)).wait_send()

    @pl.when(s == N_DEV - 1)
    def _():
        xb, yb, zb = _coords(pos)
        bits = [xb, yb, zb]
        q4 = lax.rem(pos, 4)
        partners = [
            zb * 4 + jnp.bitwise_xor(q4, 1),
            zb * 4 + (3 - q4),
            lax.rem(pos + 4, N_DEV),
        ]
        c_own = _nest(pos)

        for d in range(3):
            pl.semaphore_signal(
                ready_sem.at[d],
                inc=1,
                device_id=(partners[d],),
                device_id_type=pl.DeviceIdType.MESH,
            )
        for d in range(3):
            pl.semaphore_wait(ready_sem.at[d], 1)

        for t in range(3):
            descs = []
            for gi, (coff, cw, order) in enumerate(_AG_GROUPS):
                d = order[t]
                masks = [0]
                for td in order[:t]:
                    masks = masks + [mk ^ _FLIP[td] for mk in masks]
                for j, mk in enumerate(masks):
                    cc = jnp.bitwise_xor(c_own, mk)
                    r = cc * QBLK
                    rdma = pltpu.make_async_remote_copy(
                        src_ref=out_ref.at[pl.ds(r, QBLK), pl.ds(coff, cw)],
                        dst_ref=out_ref.at[pl.ds(r, QBLK), pl.ds(coff, cw)],
                        send_sem=ag_send.at[gi, t, j],
                        recv_sem=ag_recv.at[gi, t, j],
                        device_id=(partners[d],),
                        device_id_type=pl.DeviceIdType.MESH,
                    )
                    rdma.start()
                    descs.append(rdma)
            for rdma in descs:
                rdma.wait_recv()
                rdma.wait_send()


def kernel(x, Wq, K_ext, V_ext, Wo):
    pos = lax.axis_index("i")
    K = (
        lax.dynamic_slice_in_dim(K_ext[0], pos * H_LOCAL, H_LOCAL, axis=1)
        .reshape(SKV, H_LOCAL * DH)
        .astype(jnp.bfloat16)
    )
    V = (
        lax.dynamic_slice_in_dim(V_ext[0], pos * H_LOCAL, H_LOCAL, axis=1)
        .reshape(SKV, H_LOCAL * DH)
        .astype(jnp.bfloat16)
    )

    out = pl.pallas_call(
        _body,
        grid=(N_DEV,),
        in_specs=[
            pl.BlockSpec((SQ, D_MODEL), lambda s: (0, 0)),
            pl.BlockSpec((D_MODEL, H_LOCAL * DH), lambda s: (0, 0)),
            pl.BlockSpec((SKV, H_LOCAL * DH), lambda s: (0, 0)),
            pl.BlockSpec((SKV, H_LOCAL * DH), lambda s: (0, 0)),
            pl.BlockSpec((H_LOCAL * DH, D_MODEL), lambda s: (0, 0)),
        ],
        out_specs=pl.BlockSpec((SQ, D_MODEL), lambda s: (0, 0)),
        out_shape=jax.ShapeDtypeStruct((SQ, D_MODEL), jnp.float32),
        scratch_shapes=[
            pltpu.VMEM(((N_DEV - 1) * QBLK, D_MODEL), jnp.float32),
            pltpu.VMEM(((N_DEV - 1) * QBLK, D_MODEL), jnp.float32),
            pltpu.VMEM((QBLK, H_LOCAL * DH), jnp.bfloat16),
            pltpu.VMEM((QBLK, H_LOCAL * DH), jnp.bfloat16),
            pltpu.SemaphoreType.DMA((N_DEV - 1,)),
            pltpu.SemaphoreType.DMA((N_DEV - 1,)),
            pltpu.SemaphoreType.DMA((3, 3, 4)),
            pltpu.SemaphoreType.DMA((3, 3, 4)),
            pltpu.SemaphoreType.REGULAR((3,)),
        ],
        compiler_params=pltpu.CompilerParams(
            dimension_semantics=("arbitrary",),
            has_side_effects=True,
            vmem_limit_bytes=56 * 1024 * 1024,
        ),
    )(
        x[0].astype(jnp.bfloat16),
        Wq.astype(jnp.bfloat16),
        K,
        V,
        Wo.astype(jnp.bfloat16),
    )
    return out[None]


# device time: 184690 ns/iter; 1.0214x vs baseline; 1.0077x over previous
import jax
import jax.numpy as jnp
from jax import lax
from jax.experimental import pallas as pl
from jax.experimental.pallas import tpu as pltpu

N_DEV = 8
SQ = 2048
SKV = 2048
D_MODEL = 1024
DH = 128
H_LOCAL = 8
WIN = 128
QBLK = 256
KSPAN = 512
SCALE = 0.08838834764831843

_AG_GROUPS = (
    (0, 384, (2, 1, 0)),
    (384, 384, (0, 2, 1)),
    (768, 256, (1, 0, 2)),
)
_FLIP = (4, 2, 1)

_NO_COMM = False
_SKIP_RS = False
_SKIP_AG = False


def _coords(p):
    q = lax.rem(p, 4)
    zb = p // 4
    xb = lax.rem((q + 1) // 2, 2)
    yb = q // 2
    return xb, yb, zb


def _nest(p):
    xb, yb, zb = _coords(p)
    return 4 * xb + 2 * yb + zb


def _ring(r):
    return jnp.where(r < 4, r, 11 - r)


def _body(
    x_ref, wq_ref, k_ref, v_ref, wo_ref, out_ref,
    comm_ref, stage_ref, qbf_scr, ctx_scr,
    rs_send, rs_recv, ag_send, ag_recv, ready_sem,
):
    s = pl.program_id(0)
    pos = lax.axis_index("i")
    ridx = _ring(pos)
    right = _ring(lax.rem(ridx + 1, N_DEV))

    cs = _nest(_ring(lax.rem(ridx - s - 1 + 2 * N_DEV, N_DEV)))
    rows = cs * QBLK
    start = jnp.clip(cs * 2 - 1, 0, (SKV - KSPAN) // 128) * 128

    qbf_scr[...] = (
        jnp.dot(
            x_ref[pl.ds(rows, QBLK), :],
            wq_ref[...],
            preferred_element_type=jnp.float32,
        )
        * SCALE
    ).astype(jnp.bfloat16)

    qi = rows + lax.broadcasted_iota(jnp.int32, (QBLK, KSPAN), 0)
    ki = start + lax.broadcasted_iota(jnp.int32, (QBLK, KSPAN), 1)
    win_mask = jnp.abs(qi - ki) <= WIN

    def head(h, carry):
        qh = qbf_scr[:, pl.ds(h * DH, DH)]
        kblk = k_ref[pl.ds(start, KSPAN), pl.ds(h * DH, DH)]
        sc = lax.dot_general(
            qh, kblk, (((1,), (1,)), ((), ())),
            preferred_element_type=jnp.float32,
        )
        sc = jnp.where(win_mask, sc, -1e9)
        m = jnp.max(sc, axis=1, keepdims=True)
        w = jnp.exp(sc - m)
        denom = jnp.sum(w, axis=1, keepdims=True)
        vblk = v_ref[pl.ds(start, KSPAN), pl.ds(h * DH, DH)]
        ctx = (
            jnp.dot(
                w.astype(jnp.bfloat16), vblk,
                preferred_element_type=jnp.float32,
            )
            / denom
        )
        ctx_scr[:, pl.ds(h * DH, DH)] = ctx.astype(jnp.bfloat16)
        return carry

    lax.fori_loop(0, H_LOCAL, head, 0)

    out_ref[pl.ds(rows, QBLK), :] = jnp.dot(
        ctx_scr[...], wo_ref[...], preferred_element_type=jnp.float32
    )

    if _NO_COMM:
        return

    sm1 = jnp.maximum(s - 1, 0)

    def ring_desc(idx):
        return pltpu.make_async_remote_copy(
            src_ref=stage_ref.at[pl.ds(idx * QBLK, QBLK), :],
            dst_ref=comm_ref.at[pl.ds(idx * QBLK, QBLK), :],
            send_sem=rs_send.at[idx],
            recv_sem=rs_recv.at[idx],
            device_id=(right,),
            device_id_type=pl.DeviceIdType.MESH,
        )

    if not _SKIP_RS:
        @pl.when(s > 0)
        def _():
            ring_desc(sm1).wait_recv()
            out_ref[pl.ds(rows, QBLK), :] += comm_ref[pl.ds(sm1 * QBLK, QBLK), :]

        @pl.when(s < N_DEV - 1)
        def _():
            stage_ref[pl.ds(s * QBLK, QBLK), :] = out_ref[pl.ds(rows, QBLK), :]
            ring_desc(s).start()

        @pl.when(s > 0)
        def _():
            ring_desc(jnp.minimum(sm1, N_DEV - 2)).wait_send()

    if _SKIP_AG:
        return

    @pl.when(s == N_DEV - 1)
    def _():
        xb, yb, zb = _coords(pos)
        bits = [xb, yb, zb]
        q4 = lax.rem(pos, 4)
        partners = [
            zb * 4 + jnp.bitwise_xor(q4, 1),
            zb * 4 + (3 - q4),
            lax.rem(pos + 4, N_DEV),
        ]
        c_own = _nest(pos)

        for d in range(3):
            pl.semaphore_signal(
                ready_sem.at[d],
                inc=1,
                device_id=(partners[d],),
                device_id_type=pl.DeviceIdType.MESH,
            )
        for d in range(3):
            pl.semaphore_wait(ready_sem.at[d], 1)

        for t in range(3):
            descs = []
            for gi, (coff, cw, order) in enumerate(_AG_GROUPS):
                d = order[t]
                masks = [0]
                for td in order[:t]:
                    masks = masks + [mk ^ _FLIP[td] for mk in masks]
                for j, mk in enumerate(masks):
                    cc = jnp.bitwise_xor(c_own, mk)
                    r = cc * QBLK
                    rdma = pltpu.make_async_remote_copy(
                        src_ref=out_ref.at[pl.ds(r, QBLK), pl.ds(coff, cw)],
                        dst_ref=out_ref.at[pl.ds(r, QBLK), pl.ds(coff, cw)],
                        send_sem=ag_send.at[gi, t, j],
                        recv_sem=ag_recv.at[gi, t, j],
                        device_id=(partners[d],),
                        device_id_type=pl.DeviceIdType.MESH,
                    )
                    rdma.start()
                    descs.append(rdma)
            for rdma in descs:
                rdma.wait_recv()
                rdma.wait_send()


def kernel(x, Wq, K_ext, V_ext, Wo):
    pos = lax.axis_index("i")
    K = (
        lax.dynamic_slice_in_dim(K_ext[0], pos * H_LOCAL, H_LOCAL, axis=1)
        .reshape(SKV, H_LOCAL * DH)
        .astype(jnp.bfloat16)
    )
    V = (
        lax.dynamic_slice_in_dim(V_ext[0], pos * H_LOCAL, H_LOCAL, axis=1)
        .reshape(SKV, H_LOCAL * DH)
        .astype(jnp.bfloat16)
    )

    out = pl.pallas_call(
        _body,
        grid=(N_DEV,),
        in_specs=[
            pl.BlockSpec((SQ, D_MODEL), lambda s: (0, 0)),
            pl.BlockSpec((D_MODEL, H_LOCAL * DH), lambda s: (0, 0)),
            pl.BlockSpec((SKV, H_LOCAL * DH), lambda s: (0, 0)),
            pl.BlockSpec((SKV, H_LOCAL * DH), lambda s: (0, 0)),
            pl.BlockSpec((H_LOCAL * DH, D_MODEL), lambda s: (0, 0)),
        ],
        out_specs=pl.BlockSpec((SQ, D_MODEL), lambda s: (0, 0)),
        out_shape=jax.ShapeDtypeStruct((SQ, D_MODEL), jnp.float32),
        scratch_shapes=[
            pltpu.VMEM(((N_DEV - 1) * QBLK, D_MODEL), jnp.float32),
            pltpu.VMEM(((N_DEV - 1) * QBLK, D_MODEL), jnp.float32),
            pltpu.VMEM((QBLK, H_LOCAL * DH), jnp.bfloat16),
            pltpu.VMEM((QBLK, H_LOCAL * DH), jnp.bfloat16),
            pltpu.SemaphoreType.DMA((N_DEV - 1,)),
            pltpu.SemaphoreType.DMA((N_DEV - 1,)),
            pltpu.SemaphoreType.DMA((3, 3, 4)),
            pltpu.SemaphoreType.DMA((3, 3, 4)),
            pltpu.SemaphoreType.REGULAR((3,)),
        ],
        compiler_params=pltpu.CompilerParams(
            dimension_semantics=("arbitrary",),
            has_side_effects=True,
            vmem_limit_bytes=56 * 1024 * 1024,
        ),
    )(
        x[0].astype(jnp.bfloat16),
        Wq.astype(jnp.bfloat16),
        K,
        V,
        Wo.astype(jnp.bfloat16),
    )
    return out[None]


# device time: 167567 ns/iter; 1.1258x vs baseline; 1.1022x over previous
import jax
import jax.numpy as jnp
from jax import lax
from jax.experimental import pallas as pl
from jax.experimental.pallas import tpu as pltpu

N_DEV = 8
SQ = 2048
SKV = 2048
D_MODEL = 1024
DH = 128
H_LOCAL = 8
WIN = 128
QBLK = 256
KSPAN = 512
SCALE = 0.08838834764831843

_GROUPS = (
    (0, 768, (0, 1, 2)),
    (768, 640, (1, 2, 0)),
    (1408, 640, (2, 0, 1)),
)
_COMM_OFF = []
_off = 0
for _base, _rows, _order in _GROUPS:
    _offs = []
    for _s in range(3):
        _offs.append(_off)
        _off += _rows >> (_s + 1)
    _COMM_OFF.append(tuple(_offs))
_COMM_ROWS = _off

_NO_COMM = False


def _body(
    x_ref, wq_ref, k_ref, v_ref, wo_ref, out_ref,
    comm_ref, qbf_scr, ctx_scr,
    rs_send, rs_recv, ag_send, ag_recv, ready_sem,
):
    s = pl.program_id(0)
    pos = lax.axis_index("i")
    rows = s * QBLK
    start = jnp.clip(s * 2 - 1, 0, (SKV - KSPAN) // 128) * 128

    qbf_scr[...] = (
        jnp.dot(
            x_ref[pl.ds(rows, QBLK), :],
            wq_ref[...],
            preferred_element_type=jnp.float32,
        )
        * SCALE
    ).astype(jnp.bfloat16)

    qi = rows + lax.broadcasted_iota(jnp.int32, (QBLK, KSPAN), 0)
    ki = start + lax.broadcasted_iota(jnp.int32, (QBLK, KSPAN), 1)
    win_mask = jnp.abs(qi - ki) <= WIN

    def head(h, carry):
        qh = qbf_scr[:, pl.ds(h * DH, DH)]
        kblk = k_ref[pl.ds(start, KSPAN), pl.ds(h * DH, DH)]
        sc = lax.dot_general(
            qh, kblk, (((1,), (1,)), ((), ())),
            preferred_element_type=jnp.float32,
        )
        sc = jnp.where(win_mask, sc, -1e9)
        m = jnp.max(sc, axis=1, keepdims=True)
        w = jnp.exp(sc - m)
        denom = jnp.sum(w, axis=1, keepdims=True)
        vblk = v_ref[pl.ds(start, KSPAN), pl.ds(h * DH, DH)]
        ctx = (
            jnp.dot(
                w.astype(jnp.bfloat16), vblk,
                preferred_element_type=jnp.float32,
            )
            / denom
        )
        ctx_scr[:, pl.ds(h * DH, DH)] = ctx.astype(jnp.bfloat16)
        return carry

    lax.fori_loop(0, H_LOCAL, head, 0)

    out_ref[pl.ds(rows, QBLK), :] = jnp.dot(
        ctx_scr[...], wo_ref[...], preferred_element_type=jnp.float32
    )

    if _NO_COMM:
        return

    @pl.when(s == N_DEV - 1)
    def _():
        q4 = lax.rem(pos, 4)
        zb = pos // 4
        xb = lax.rem((q4 + 1) // 2, 2)
        yb = q4 // 2
        partners = [
            zb * 4 + jnp.bitwise_xor(q4, 1),
            zb * 4 + (3 - q4),
            lax.rem(pos + 4, N_DEV),
        ]
        bits = [xb, yb, zb]

        keep = [jnp.int32(g[0]) for g in _GROUPS]
        for st in range(3):
            rdmas = []
            for gi, (base, grows, order) in enumerate(_GROUPS):
                size = grows >> (st + 1)
                d = order[st]
                b = bits[d]
                send_start = keep[gi] + (1 - b) * size
                keep[gi] = keep[gi] + b * size
                rdma = pltpu.make_async_remote_copy(
                    src_ref=out_ref.at[pl.ds(send_start, size), :],
                    dst_ref=comm_ref.at[pl.ds(_COMM_OFF[gi][st], size), :],
                    send_sem=rs_send.at[gi, st],
                    recv_sem=rs_recv.at[gi, st],
                    device_id=(partners[d],),
                    device_id_type=pl.DeviceIdType.MESH,
                )
                rdma.start()
                rdmas.append(rdma)
            for gi, (base, grows, order) in enumerate(_GROUPS):
                size = grows >> (st + 1)
                rdmas[gi].wait_recv()
                out_ref[pl.ds(keep[gi], size), :] += comm_ref[
                    pl.ds(_COMM_OFF[gi][st], size), :
                ]
                rdmas[gi].wait_send()

        for d in range(3):
            pl.semaphore_signal(
                ready_sem.at[d],
                inc=1,
                device_id=(partners[d],),
                device_id_type=pl.DeviceIdType.MESH,
            )
        for d in range(3):
            pl.semaphore_wait(ready_sem.at[d], 1)

        cur = keep
        for st in (2, 1, 0):
            rdmas = []
            for gi, (base, grows, order) in enumerate(_GROUPS):
                size = grows >> (st + 1)
                rdma = pltpu.make_async_remote_copy(
                    src_ref=out_ref.at[pl.ds(cur[gi], size), :],
                    dst_ref=out_ref.at[pl.ds(cur[gi], size), :],
                    send_sem=ag_send.at[gi, st],
                    recv_sem=ag_recv.at[gi, st],
                    device_id=(partners[order[st]],),
                    device_id_type=pl.DeviceIdType.MESH,
                )
                rdma.start()
                rdmas.append(rdma)
            for gi, (base, grows, order) in enumerate(_GROUPS):
                size = grows >> (st + 1)
                rdmas[gi].wait_recv()
                rdmas[gi].wait_send()
                cur[gi] = cur[gi] - bits[order[st]] * size


def kernel(x, Wq, K_ext, V_ext, Wo):
    pos = lax.axis_index("i")
    K = (
        lax.dynamic_slice_in_dim(K_ext[0], pos * H_LOCAL, H_LOCAL, axis=1)
        .reshape(SKV, H_LOCAL * DH)
        .astype(jnp.bfloat16)
    )
    V = (
        lax.dynamic_slice_in_dim(V_ext[0], pos * H_LOCAL, H_LOCAL, axis=1)
        .reshape(SKV, H_LOCAL * DH)
        .astype(jnp.bfloat16)
    )

    out = pl.pallas_call(
        _body,
        grid=(N_DEV,),
        in_specs=[
            pl.BlockSpec((SQ, D_MODEL), lambda s: (0, 0)),
            pl.BlockSpec((D_MODEL, H_LOCAL * DH), lambda s: (0, 0)),
            pl.BlockSpec((SKV, H_LOCAL * DH), lambda s: (0, 0)),
            pl.BlockSpec((SKV, H_LOCAL * DH), lambda s: (0, 0)),
            pl.BlockSpec((H_LOCAL * DH, D_MODEL), lambda s: (0, 0)),
        ],
        out_specs=pl.BlockSpec((SQ, D_MODEL), lambda s: (0, 0)),
        out_shape=jax.ShapeDtypeStruct((SQ, D_MODEL), jnp.float32),
        scratch_shapes=[
            pltpu.VMEM((_COMM_ROWS, D_MODEL), jnp.float32),
            pltpu.VMEM((QBLK, H_LOCAL * DH), jnp.bfloat16),
            pltpu.VMEM((QBLK, H_LOCAL * DH), jnp.bfloat16),
            pltpu.SemaphoreType.DMA((3, 3)),
            pltpu.SemaphoreType.DMA((3, 3)),
            pltpu.SemaphoreType.DMA((3, 3)),
            pltpu.SemaphoreType.DMA((3, 3)),
            pltpu.SemaphoreType.REGULAR((3,)),
        ],
        compiler_params=pltpu.CompilerParams(
            dimension_semantics=("arbitrary",),
            has_side_effects=True,
            vmem_limit_bytes=56 * 1024 * 1024,
        ),
    )(
        x[0].astype(jnp.bfloat16),
        Wq.astype(jnp.bfloat16),
        K,
        V,
        Wo.astype(jnp.bfloat16),
    )
    return out[None]


# device time: 164496 ns/iter; 1.1468x vs baseline; 1.0187x over previous
import jax
import jax.numpy as jnp
from jax import lax
from jax.experimental import pallas as pl
from jax.experimental.pallas import tpu as pltpu

N_DEV = 8
SQ = 2048
SKV = 2048
D_MODEL = 1024
DH = 128
H_LOCAL = 8
WIN = 128
QBLK = 256
KSPAN = 512
SCALE = 0.08838834764831843

_GROUPS = (
    (0, 768, (0, 1, 2)),
    (768, 640, (1, 2, 0)),
    (1408, 640, (2, 0, 1)),
)
_COMM_OFF = []
_off = 0
for _base, _rows, _order in _GROUPS:
    _offs = []
    for _s in range(3):
        _offs.append(_off)
        _off += _rows >> (_s + 1)
    _COMM_OFF.append(tuple(_offs))
_COMM_ROWS = _off

_NO_COMM = False


def _body(
    x_ref, wq_ref, k_ref, v_ref, wo_ref, out_ref,
    comm_ref, qbf_scr, ctx_scr,
    rs_send, rs_recv, ag_send, ag_recv, ready_sem,
):
    s = pl.program_id(0)
    pos = lax.axis_index("i")
    rows = s * QBLK
    start = jnp.clip(s * 2 - 1, 0, (SKV - KSPAN) // 128) * 128

    qbf_scr[...] = (
        jnp.dot(
            x_ref[pl.ds(rows, QBLK), :],
            wq_ref[...],
            preferred_element_type=jnp.float32,
        )
        * SCALE
    ).astype(jnp.bfloat16)

    qi = rows + lax.broadcasted_iota(jnp.int32, (QBLK, KSPAN), 0)
    ki = start + lax.broadcasted_iota(jnp.int32, (QBLK, KSPAN), 1)
    win_mask = jnp.abs(qi - ki) <= WIN

    def head(h, carry):
        qh = qbf_scr[:, pl.ds(h * DH, DH)]
        kblk = k_ref[pl.ds(start, KSPAN), pl.ds(h * DH, DH)]
        sc = lax.dot_general(
            qh, kblk, (((1,), (1,)), ((), ())),
            preferred_element_type=jnp.float32,
        )
        w = jnp.where(win_mask, jnp.exp(sc), 0.0)
        denom = jnp.sum(w, axis=1, keepdims=True)
        vblk = v_ref[pl.ds(start, KSPAN), pl.ds(h * DH, DH)]
        ctx = (
            jnp.dot(
                w.astype(jnp.bfloat16), vblk,
                preferred_element_type=jnp.float32,
            )
            / denom
        )
        ctx_scr[:, pl.ds(h * DH, DH)] = ctx.astype(jnp.bfloat16)
        return carry

    lax.fori_loop(0, H_LOCAL, head, 0)

    out_ref[pl.ds(rows, QBLK), :] = jnp.dot(
        ctx_scr[...], wo_ref[...], preferred_element_type=jnp.float32
    )

    if _NO_COMM:
        return

    @pl.when(s == N_DEV - 1)
    def _():
        q4 = lax.rem(pos, 4)
        zb = pos // 4
        xb = lax.rem((q4 + 1) // 2, 2)
        yb = q4 // 2
        partners = [
            zb * 4 + jnp.bitwise_xor(q4, 1),
            zb * 4 + (3 - q4),
            lax.rem(pos + 4, N_DEV),
        ]
        bits = [xb, yb, zb]

        keep = [jnp.int32(g[0]) for g in _GROUPS]
        for st in range(3):
            rdmas = []
            for gi, (base, grows, order) in enumerate(_GROUPS):
                size = grows >> (st + 1)
                d = order[st]
                b = bits[d]
                send_start = keep[gi] + (1 - b) * size
                keep[gi] = keep[gi] + b * size
                rdma = pltpu.make_async_remote_copy(
                    src_ref=out_ref.at[pl.ds(send_start, size), :],
                    dst_ref=comm_ref.at[pl.ds(_COMM_OFF[gi][st], size), :],
                    send_sem=rs_send.at[gi, st],
                    recv_sem=rs_recv.at[gi, st],
                    device_id=(partners[d],),
                    device_id_type=pl.DeviceIdType.MESH,
                )
                rdma.start()
                rdmas.append(rdma)
            for gi, (base, grows, order) in enumerate(_GROUPS):
                size = grows >> (st + 1)
                rdmas[gi].wait_recv()
                out_ref[pl.ds(keep[gi], size), :] += comm_ref[
                    pl.ds(_COMM_OFF[gi][st], size), :
                ]
                rdmas[gi].wait_send()

        for d in range(3):
            pl.semaphore_signal(
                ready_sem.at[d],
                inc=1,
                device_id=(partners[d],),
                device_id_type=pl.DeviceIdType.MESH,
            )
        for d in range(3):
            pl.semaphore_wait(ready_sem.at[d], 1)

        cur = keep
        for st in (2, 1, 0):
            rdmas = []
            for gi, (base, grows, order) in enumerate(_GROUPS):
                size = grows >> (st + 1)
                rdma = pltpu.make_async_remote_copy(
                    src_ref=out_ref.at[pl.ds(cur[gi], size), :],
                    dst_ref=out_ref.at[pl.ds(cur[gi], size), :],
                    send_sem=ag_send.at[gi, st],
                    recv_sem=ag_recv.at[gi, st],
                    device_id=(partners[order[st]],),
                    device_id_type=pl.DeviceIdType.MESH,
                )
                rdma.start()
                rdmas.append(rdma)
            for gi, (base, grows, order) in enumerate(_GROUPS):
                size = grows >> (st + 1)
                rdmas[gi].wait_recv()
                rdmas[gi].wait_send()
                cur[gi] = cur[gi] - bits[order[st]] * size


def kernel(x, Wq, K_ext, V_ext, Wo):
    pos = lax.axis_index("i")
    K = (
        lax.dynamic_slice_in_dim(K_ext[0], pos * H_LOCAL, H_LOCAL, axis=1)
        .reshape(SKV, H_LOCAL * DH)
        .astype(jnp.bfloat16)
    )
    V = (
        lax.dynamic_slice_in_dim(V_ext[0], pos * H_LOCAL, H_LOCAL, axis=1)
        .reshape(SKV, H_LOCAL * DH)
        .astype(jnp.bfloat16)
    )

    out = pl.pallas_call(
        _body,
        grid=(N_DEV,),
        in_specs=[
            pl.BlockSpec((SQ, D_MODEL), lambda s: (0, 0)),
            pl.BlockSpec((D_MODEL, H_LOCAL * DH), lambda s: (0, 0)),
            pl.BlockSpec((SKV, H_LOCAL * DH), lambda s: (0, 0)),
            pl.BlockSpec((SKV, H_LOCAL * DH), lambda s: (0, 0)),
            pl.BlockSpec((H_LOCAL * DH, D_MODEL), lambda s: (0, 0)),
        ],
        out_specs=pl.BlockSpec((SQ, D_MODEL), lambda s: (0, 0)),
        out_shape=jax.ShapeDtypeStruct((SQ, D_MODEL), jnp.float32),
        scratch_shapes=[
            pltpu.VMEM((_COMM_ROWS, D_MODEL), jnp.float32),
            pltpu.VMEM((QBLK, H_LOCAL * DH), jnp.bfloat16),
            pltpu.VMEM((QBLK, H_LOCAL * DH), jnp.bfloat16),
            pltpu.SemaphoreType.DMA((3, 3)),
            pltpu.SemaphoreType.DMA((3, 3)),
            pltpu.SemaphoreType.DMA((3, 3)),
            pltpu.SemaphoreType.DMA((3, 3)),
            pltpu.SemaphoreType.REGULAR((3,)),
        ],
        compiler_params=pltpu.CompilerParams(
            dimension_semantics=("arbitrary",),
            has_side_effects=True,
            vmem_limit_bytes=56 * 1024 * 1024,
        ),
    )(
        x[0].astype(jnp.bfloat16),
        Wq.astype(jnp.bfloat16),
        K,
        V,
        Wo.astype(jnp.bfloat16),
    )
    return out[None]


# device time: 161675 ns/iter; 1.1668x vs baseline; 1.0174x over previous
import jax
import jax.numpy as jnp
from jax import lax
from jax.experimental import pallas as pl
from jax.experimental.pallas import tpu as pltpu

N_DEV = 8
SQ = 2048
SKV = 2048
D_MODEL = 1024
DH = 128
H_LOCAL = 8
WIN = 128
QBLK = 256
KSPAN = 512
SCALE = 0.08838834764831843

_GROUPS = (
    (0, 768, (0, 1, 2)),
    (768, 640, (1, 2, 0)),
    (1408, 640, (2, 0, 1)),
)
_COMM_OFF = []
_off = 0
for _base, _rows, _order in _GROUPS:
    _offs = []
    for _s in range(3):
        _offs.append(_off)
        _off += _rows >> (_s + 1)
    _COMM_OFF.append(tuple(_offs))
_COMM_ROWS = _off

_NO_COMM = False


def _body(
    x_ref, wq_ref, k_ref, v_ref, wo_ref, out_ref,
    comm_ref, qbf_scr, ctx_scr,
    rs_send, rs_recv, ag_send, ag_recv, ready_sem,
):
    s = pl.program_id(0)
    pos = lax.axis_index("i")
    rows = s * QBLK
    start = jnp.clip(s * 2 - 1, 0, (SKV - KSPAN) // 128) * 128

    qbf_scr[...] = (
        jnp.dot(
            x_ref[pl.ds(rows, QBLK), :],
            wq_ref[...],
            preferred_element_type=jnp.float32,
        )
        * SCALE
    ).astype(jnp.bfloat16)

    qi = rows + lax.broadcasted_iota(jnp.int32, (QBLK, KSPAN), 0)
    ki = start + lax.broadcasted_iota(jnp.int32, (QBLK, KSPAN), 1)
    win_mask = jnp.abs(qi - ki) <= WIN

    def head(h, carry):
        qh = qbf_scr[:, pl.ds(h * DH, DH)]
        kblk = k_ref[pl.ds(start, KSPAN), pl.ds(h * DH, DH)]
        sc = lax.dot_general(
            qh, kblk, (((1,), (1,)), ((), ())),
            preferred_element_type=jnp.float32,
        )
        w = jnp.where(win_mask, jnp.exp(sc), 0.0)
        denom = jnp.sum(w, axis=1, keepdims=True)
        vblk = v_ref[pl.ds(start, KSPAN), pl.ds(h * DH, DH)]
        ctx = (
            jnp.dot(
                w.astype(jnp.bfloat16), vblk,
                preferred_element_type=jnp.float32,
            )
            / denom
        )
        ctx_scr[:, pl.ds(h * DH, DH)] = ctx.astype(jnp.bfloat16)
        return carry

    lax.fori_loop(0, H_LOCAL, head, 0)

    out_ref[pl.ds(rows, QBLK), :] = jnp.dot(
        ctx_scr[...], wo_ref[...], preferred_element_type=jnp.float32
    )

    if _NO_COMM:
        return

    q4 = lax.rem(pos, 4)
    zb = pos // 4
    xb = lax.rem((q4 + 1) // 2, 2)
    yb = q4 // 2
    partners = [
        zb * 4 + jnp.bitwise_xor(q4, 1),
        zb * 4 + (3 - q4),
        lax.rem(pos + 4, N_DEV),
    ]
    bits = [xb, yb, zb]

    def rs_stage0_desc(gi):
        base, grows, order = _GROUPS[gi]
        size = grows >> 1
        b = bits[order[0]]
        return pltpu.make_async_remote_copy(
            src_ref=out_ref.at[pl.ds(base + (1 - b) * size, size), :],
            dst_ref=comm_ref.at[pl.ds(_COMM_OFF[gi][0], size), :],
            send_sem=rs_send.at[gi, 0],
            recv_sem=rs_recv.at[gi, 0],
            device_id=(partners[order[0]],),
            device_id_type=pl.DeviceIdType.MESH,
        )

    @pl.when(s == N_DEV - 2)
    def _():
        rs_stage0_desc(0).start()
        rs_stage0_desc(1).start()

    @pl.when(s == N_DEV - 1)
    def _():
        rs_stage0_desc(2).start()
        keep = []
        for gi, (base, grows, order) in enumerate(_GROUPS):
            size = grows >> 1
            keep.append(base + bits[order[0]] * size)
            rdma = rs_stage0_desc(gi)
            rdma.wait_recv()
            out_ref[pl.ds(keep[gi], size), :] += comm_ref[
                pl.ds(_COMM_OFF[gi][0], size), :
            ]
            rdma.wait_send()
        for st in (1, 2):
            rdmas = []
            for gi, (base, grows, order) in enumerate(_GROUPS):
                size = grows >> (st + 1)
                d = order[st]
                b = bits[d]
                send_start = keep[gi] + (1 - b) * size
                keep[gi] = keep[gi] + b * size
                rdma = pltpu.make_async_remote_copy(
                    src_ref=out_ref.at[pl.ds(send_start, size), :],
                    dst_ref=comm_ref.at[pl.ds(_COMM_OFF[gi][st], size), :],
                    send_sem=rs_send.at[gi, st],
                    recv_sem=rs_recv.at[gi, st],
                    device_id=(partners[d],),
                    device_id_type=pl.DeviceIdType.MESH,
                )
                rdma.start()
                rdmas.append(rdma)
            for gi, (base, grows, order) in enumerate(_GROUPS):
                size = grows >> (st + 1)
                rdmas[gi].wait_recv()
                out_ref[pl.ds(keep[gi], size), :] += comm_ref[
                    pl.ds(_COMM_OFF[gi][st], size), :
                ]
                rdmas[gi].wait_send()

        for d in range(3):
            pl.semaphore_signal(
                ready_sem.at[d],
                inc=1,
                device_id=(partners[d],),
                device_id_type=pl.DeviceIdType.MESH,
            )
        for d in range(3):
            pl.semaphore_wait(ready_sem.at[d], 1)

        cur = keep
        for st in (2, 1, 0):
            rdmas = []
            for gi, (base, grows, order) in enumerate(_GROUPS):
                size = grows >> (st + 1)
                rdma = pltpu.make_async_remote_copy(
                    src_ref=out_ref.at[pl.ds(cur[gi], size), :],
                    dst_ref=out_ref.at[pl.ds(cur[gi], size), :],
                    send_sem=ag_send.at[gi, st],
                    recv_sem=ag_recv.at[gi, st],
                    device_id=(partners[order[st]],),
                    device_id_type=pl.DeviceIdType.MESH,
                )
                rdma.start()
                rdmas.append(rdma)
            for gi, (base, grows, order) in enumerate(_GROUPS):
                size = grows >> (st + 1)
                rdmas[gi].wait_recv()
                rdmas[gi].wait_send()
                cur[gi] = cur[gi] - bits[order[st]] * size


def kernel(x, Wq, K_ext, V_ext, Wo):
    pos = lax.axis_index("i")
    K = (
        lax.dynamic_slice_in_dim(K_ext[0], pos * H_LOCAL, H_LOCAL, axis=1)
        .reshape(SKV, H_LOCAL * DH)
        .astype(jnp.bfloat16)
    )
    V = (
        lax.dynamic_slice_in_dim(V_ext[0], pos * H_LOCAL, H_LOCAL, axis=1)
        .reshape(SKV, H_LOCAL * DH)
        .astype(jnp.bfloat16)
    )

    out = pl.pallas_call(
        _body,
        grid=(N_DEV,),
        in_specs=[
            pl.BlockSpec((SQ, D_MODEL), lambda s: (0, 0)),
            pl.BlockSpec((D_MODEL, H_LOCAL * DH), lambda s: (0, 0)),
            pl.BlockSpec((SKV, H_LOCAL * DH), lambda s: (0, 0)),
            pl.BlockSpec((SKV, H_LOCAL * DH), lambda s: (0, 0)),
            pl.BlockSpec((H_LOCAL * DH, D_MODEL), lambda s: (0, 0)),
        ],
        out_specs=pl.BlockSpec((SQ, D_MODEL), lambda s: (0, 0)),
        out_shape=jax.ShapeDtypeStruct((SQ, D_MODEL), jnp.float32),
        scratch_shapes=[
            pltpu.VMEM((_COMM_ROWS, D_MODEL), jnp.float32),
            pltpu.VMEM((QBLK, H_LOCAL * DH), jnp.bfloat16),
            pltpu.VMEM((QBLK, H_LOCAL * DH), jnp.bfloat16),
            pltpu.SemaphoreType.DMA((3, 3)),
            pltpu.SemaphoreType.DMA((3, 3)),
            pltpu.SemaphoreType.DMA((3, 3)),
            pltpu.SemaphoreType.DMA((3, 3)),
            pltpu.SemaphoreType.REGULAR((3,)),
        ],
        compiler_params=pltpu.CompilerParams(
            dimension_semantics=("arbitrary",),
            has_side_effects=True,
            vmem_limit_bytes=56 * 1024 * 1024,
        ),
    )(
        x[0].astype(jnp.bfloat16),
        Wq.astype(jnp.bfloat16),
        K,
        V,
        Wo.astype(jnp.bfloat16),
    )
    return out[None]
